# baseline (speedup 1.0000x reference)
import jax
import jax.numpy as jnp
from jax.experimental import pallas as pl
from jax.experimental.pallas import tpu as pltpu

_K = 9
_D = 2


def _copy_kernel(*refs):
    out_ref = refs[_K]
    j = pl.program_id(1)
    for k in range(_K):

        @pl.when(j == k)
        def _(k=k):
            out_ref[...] = refs[k][...]


def _slab_spec(k, b, n):
    return pl.BlockSpec((1, 1, b, n), lambda i, j, _k=k: (i, _D * _k, 0, 0))


def kernel(edge_index):
    two, b, n, kd = edge_index.shape
    t = jnp.transpose(edge_index, (0, 3, 1, 2))
    out_t = pl.pallas_call(
        _copy_kernel,
        grid=(two, _K),
        in_specs=[_slab_spec(k, b, n) for k in range(_K)],
        out_specs=pl.BlockSpec((1, 1, b, n), lambda i, j: (i, j, 0, 0)),
        out_shape=jax.ShapeDtypeStruct((two, _K, b, n), edge_index.dtype),
    )(*([t] * _K))
    return jnp.transpose(out_t, (0, 2, 3, 1))


# speedup vs baseline: 2.3580x; 2.3580x over previous
import jax
import jax.numpy as jnp
from jax.experimental import pallas as pl

_K = 9
_D = 2


def _copy_kernel(*refs):
    out_ref = refs[_K]
    for k in range(_K):
        out_ref[:, k : k + 1, :, :] = refs[k][...]


def _slab_spec(k, b, n):
    return pl.BlockSpec((1, 1, b, n), lambda i, _k=k: (i, _D * _k, 0, 0))


def kernel(edge_index):
    two, b, n, kd = edge_index.shape
    t = jnp.transpose(edge_index, (0, 3, 1, 2))
    out_t = pl.pallas_call(
        _copy_kernel,
        grid=(two,),
        in_specs=[_slab_spec(k, b, n) for k in range(_K)],
        out_specs=pl.BlockSpec((1, _K, b, n), lambda i: (i, 0, 0, 0)),
        out_shape=jax.ShapeDtypeStruct((two, _K, b, n), edge_index.dtype),
    )(*([t] * _K))
    return jnp.transpose(out_t, (0, 2, 3, 1))
